# baseline (speedup 1.0000x reference)
import functools

import jax
import jax.numpy as jnp
import numpy as np
from jax import lax
from jax.experimental import pallas as pl
from jax.experimental.pallas import tpu as pltpu
from jax.experimental.pallas import tpu_sc as plsc

B, L, D = 4096, 200, 64
NW = 32
BPW = B // NW
BB = 64


def _tables():
    pos = np.arange(L, dtype=np.float64)[:, None]
    j = np.arange(D, dtype=np.float64)[None, :]
    t = pos / np.power(10000.0, 2.0 * (np.floor(j / 2.0)) / D)
    t[:, 0::2] = np.sin(t[:, 0::2])
    t[:, 1::2] = np.cos(t[:, 1::2])
    table = t.astype(np.float32)
    cat = np.concatenate([table[::-1], table[1:], table[199:200]], axis=0)
    catp = np.concatenate([cat[:-1], cat[1:]], axis=1)
    cat_even = catp[0::2]
    cat_odd = np.concatenate(
        [catp[1::2], np.zeros((1, 2 * D), np.float32)], axis=0)
    return jnp.asarray(cat_even), jnp.asarray(cat_odd)


def _sc_starts(mask):
    mesh = plsc.VectorSubcoreMesh(core_axis_name="c", subcore_axis_name="s")

    @functools.partial(
        pl.kernel,
        mesh=mesh,
        out_type=jax.ShapeDtypeStruct((B,), jnp.int32),
        scratch_types=[
            pltpu.VMEM((BPW * L,), jnp.int32),
            pltpu.VMEM((BPW,), jnp.int32),
            pltpu.VMEM((16,), jnp.int32),
        ],
    )
    def k(mask_hbm, out_hbm, mask_v, starts_v, acc_v):
        wid = lax.axis_index("s") * 2 + lax.axis_index("c")
        base = wid * BPW
        pltpu.sync_copy(mask_hbm.at[pl.ds(base * L, BPW * L)], mask_v)
        lane = lax.iota(jnp.int32, 16)

        for g in range(BPW // 16):

            def body(r, vec, g=g):
                off = (g * 16 + r) * L
                acc = mask_v[pl.ds(off, 16)]
                for c in range(1, 12):
                    acc = acc + mask_v[pl.ds(off + 16 * c, 16)]
                tail = mask_v[pl.ds(off + L - 16, 16)]
                acc = acc + jnp.where(lane >= 8, tail, 0)
                total = acc[0]
                for i in range(1, 16):
                    total = total + acc[i]
                return jnp.where(lane == r, L - total, vec)

            vec = lax.fori_loop(0, 16, body, jnp.zeros((16,), jnp.int32))
            starts_v[pl.ds(g * 16, 16)] = vec
        pltpu.sync_copy(starts_v, out_hbm.at[pl.ds(base, BPW)])

    return k(mask.reshape(B * L))


def _tc_body(starts_ref, x_ref, cat_e_ref, cat_o_ref, out_ref):
    i = pl.program_id(0)
    for h in range(2):
        for b in range(BB):
            s = starts_ref[h * (B // 2) + i * BB + b]
            q = s >> 1
            pe_e = cat_e_ref[pl.ds(q, 100), :]
            pe_o = cat_o_ref[pl.ds(q, 100), :]
            pe = jnp.where((s & 1) == 1, pe_o, pe_e)
            out_ref[h, b] = x_ref[h, b] + pe


def kernel(x, mask):
    cat_even, cat_odd = _tables()
    starts = _sc_starts(mask)
    xv = x.reshape(2, B // 2, L * D // 128, 128)
    out = pl.pallas_call(
        _tc_body,
        grid_spec=pltpu.PrefetchScalarGridSpec(
            num_scalar_prefetch=1,
            grid=(B // 2 // BB,),
            in_specs=[
                pl.BlockSpec((2, BB, 100, 128), lambda i, st: (0, i, 0, 0)),
                pl.BlockSpec((200, 128), lambda i, st: (0, 0)),
                pl.BlockSpec((200, 128), lambda i, st: (0, 0)),
            ],
            out_specs=pl.BlockSpec(
                (2, BB, 100, 128), lambda i, st: (0, i, 0, 0)),
        ),
        out_shape=jax.ShapeDtypeStruct((2, B // 2, 100, 128), jnp.float32),
        compiler_params=pltpu.CompilerParams(
            dimension_semantics=("parallel",)),
    )(starts, xv, cat_even, cat_odd)
    return out.reshape(B, L, D)


# speedup vs baseline: 2.1193x; 2.1193x over previous
import functools

import jax
import jax.numpy as jnp
import numpy as np
from jax import lax
from jax.experimental import pallas as pl
from jax.experimental.pallas import tpu as pltpu
from jax.experimental.pallas import tpu_sc as plsc

B, L, D = 4096, 200, 64
NW = 32
BPW = B // NW
BB = 256


def _tables():
    pos = np.arange(L, dtype=np.float64)[:, None]
    j = np.arange(D, dtype=np.float64)[None, :]
    t = pos / np.power(10000.0, 2.0 * (np.floor(j / 2.0)) / D)
    t[:, 0::2] = np.sin(t[:, 0::2])
    t[:, 1::2] = np.cos(t[:, 1::2])
    table = t.astype(np.float32)
    cat = np.concatenate([table[::-1], table[1:], table[199:200]], axis=0)
    catp = np.concatenate([cat[:-1], cat[1:]], axis=1)
    cat_even = catp[0::2]
    cat_odd = np.concatenate(
        [catp[1::2], np.zeros((1, 2 * D), np.float32)], axis=0)
    return jnp.asarray(cat_even), jnp.asarray(cat_odd)


def _sc_starts(mask):
    mesh = plsc.VectorSubcoreMesh(core_axis_name="c", subcore_axis_name="s")

    @functools.partial(
        pl.kernel,
        mesh=mesh,
        out_type=jax.ShapeDtypeStruct((B,), jnp.int32),
        scratch_types=[
            pltpu.VMEM((BPW * L,), jnp.int32),
            pltpu.VMEM((BPW,), jnp.int32),
            pltpu.VMEM((16,), jnp.int32),
        ],
    )
    def k(mask_hbm, out_hbm, mask_v, starts_v, acc_v):
        wid = lax.axis_index("s") * 2 + lax.axis_index("c")
        base = wid * BPW
        pltpu.sync_copy(mask_hbm.at[pl.ds(base * L, BPW * L)], mask_v)
        lane = lax.iota(jnp.int32, 16)

        for g in range(BPW // 16):

            def body(r, vec, g=g):
                off = (g * 16 + r) * L
                acc = mask_v[pl.ds(off, 16)]
                for c in range(1, 12):
                    acc = acc + mask_v[pl.ds(off + 16 * c, 16)]
                tail = mask_v[pl.ds(off + L - 16, 16)]
                acc = acc + jnp.where(lane >= 8, tail, 0)
                total = acc[0]
                for i in range(1, 16):
                    total = total + acc[i]
                return jnp.where(lane == r, L - total, vec)

            vec = lax.fori_loop(0, 16, body, jnp.zeros((16,), jnp.int32))
            starts_v[pl.ds(g * 16, 16)] = vec
        pltpu.sync_copy(starts_v, out_hbm.at[pl.ds(base, BPW)])

    return k(mask.reshape(B * L))


def _tc_body(starts_ref, x_ref, cat_e_ref, cat_o_ref, out_ref):
    i = pl.program_id(0)
    for b in range(BB):
        s = starts_ref[i * BB + b]
        q = s >> 1
        pe_e = cat_e_ref[pl.ds(q, 100), :]
        pe_o = cat_o_ref[pl.ds(q, 100), :]
        pe = jnp.where((s & 1) == 1, pe_o, pe_e)
        out_ref[b] = x_ref[b] + pe


def kernel(x, mask):
    cat_even, cat_odd = _tables()
    starts = _sc_starts(mask)
    xv = x.reshape(B, L * D // 128, 128)
    out = pl.pallas_call(
        _tc_body,
        grid_spec=pltpu.PrefetchScalarGridSpec(
            num_scalar_prefetch=1,
            grid=(B // BB,),
            in_specs=[
                pl.BlockSpec((BB, 100, 128), lambda i, st: (i, 0, 0)),
                pl.BlockSpec((200, 128), lambda i, st: (0, 0)),
                pl.BlockSpec((200, 128), lambda i, st: (0, 0)),
            ],
            out_specs=pl.BlockSpec((BB, 100, 128), lambda i, st: (i, 0, 0)),
        ),
        out_shape=jax.ShapeDtypeStruct((B, 100, 128), jnp.float32),
        compiler_params=pltpu.CompilerParams(
            dimension_semantics=("parallel",)),
    )(starts, xv, cat_even, cat_odd)
    return out.reshape(B, L, D)
